# baseline (speedup 1.0000x reference)
import jax
import jax.numpy as jnp
from jax import lax
from jax.experimental import pallas as pl
from jax.experimental.pallas import tpu as pltpu
from jax.experimental.pallas import tpu_sc as plsc

N = 320000
D = 128
C = 10000

NC = 2
NS = 16
HALF = D // NC
ROWS_PER_SUB = N // NS
CHUNK = 160
NCHUNK = ROWS_PER_SUB // CHUNK
GROUPS = CHUNK // 16
STAGE = 128
W = 2 * HALF + 16
TRASH = 8
CROWS = C // NS


def _ds16(c):
    return pl.ds(16 * c, 16)


def _tree(vals):
    while len(vals) > 1:
        nxt = [a + b for a, b in zip(vals[::2], vals[1::2])]
        if len(vals) % 2:
            nxt[-1] = nxt[-1] + vals[-1]
        vals = nxt
    return vals[0]


def _sc_body(inputs_hbm, labels_hbm, total_hbm, total2_hbm, counts_hbm,
             idx0, idx1, x0, x1, stage, sidx,
             sem_x0, sem_x1, sem_i0, sem_i1, acc):
    ci = lax.axis_index("c")
    si = lax.axis_index("s")

    zeros16 = jnp.zeros((16,), jnp.float32)
    lane0 = lax.iota(jnp.int32, 16) == 0
    one_pat = jnp.where(lane0, 1.0, 0.0)
    sixteen_pat = jnp.where(lane0, 16.0, 0.0)
    trash_pat = jnp.full((16,), C, jnp.int32) + si % TRASH

    def sidx_trash():
        @plsc.parallel_loop(0, STAGE // 16, unroll=8)
        def _zi(k):
            sidx[_ds16(k)] = trash_pat

    def flush_scatter():
        pltpu.sync_copy(stage, acc.at[sidx], add=True)
        sidx_trash()

    @plsc.parallel_loop(0, STAGE, unroll=8)
    def _zs(r):
        for c in range(W // 16):
            stage[r, _ds16(c)] = zeros16

    sidx_trash()

    base = si * CROWS
    for z in range(CROWS // STAGE):
        pltpu.sync_copy(stage, acc.at[pl.ds(base + z * STAGE, STAGE)])
    rem = CROWS - (CROWS // STAGE) * STAGE
    zb = base + (CROWS // STAGE) * STAGE
    pltpu.sync_copy(stage.at[pl.ds(0, rem)], acc.at[pl.ds(zb, rem)])

    @pl.when(si == NS - 1)
    def _():
        pltpu.sync_copy(stage.at[pl.ds(0, TRASH)], acc.at[pl.ds(C, TRASH)])

    plsc.subcore_barrier()

    def x_src(j):
        row0 = si * ROWS_PER_SUB + j * CHUNK
        return inputs_hbm.at[pl.ds(row0, CHUNK), pl.ds(ci * HALF, HALF)]

    def issue(j, x_v, idx_v, sem_x, sem_i):
        pltpu.async_copy(x_src(j), x_v, sem_x)
        pltpu.async_copy(labels_hbm.at[si, j], idx_v, sem_i)

    def wait(j, x_v, idx_v, sem_x, sem_i):
        pltpu.make_async_copy(x_src(j), x_v, sem_x).wait()
        pltpu.make_async_copy(labels_hbm.at[si, j], idx_v, sem_i).wait()

    def process(j, x_v, idx_v, fc):
        def do_flush(f):
            flush_scatter()
            return jnp.int32(0)

        def group(g, fc):
            fc = lax.cond(fc > STAGE - 16, do_flush, lambda f: f, fc)
            r0 = 16 * g
            lv = idx_v[pl.ds(r0, 16)]

            def uniform(fc):
                for c in range(HALF // 16):
                    vs = [x_v[r0 + r, _ds16(c)] for r in range(16)]
                    sq = [v * v for v in vs]
                    stage[fc, _ds16(c)] = _tree(vs)
                    stage[fc, _ds16(4 + c)] = _tree(sq)
                stage[fc, _ds16(8)] = sixteen_pat
                old = sidx[pl.ds(fc, 16)]
                sidx[pl.ds(fc, 16)] = jnp.where(lane0, lv, old)
                return fc + 1

            def mixed(fc):
                def mrow(q, _):
                    for r4 in range(4):
                        rr = 4 * q + r4
                        k = fc + rr
                        for c in range(HALF // 16):
                            v = x_v[r0 + rr, _ds16(c)]
                            stage[k, _ds16(c)] = v
                            stage[k, _ds16(4 + c)] = v * v
                        stage[k, _ds16(8)] = one_pat
                    return 0
                lax.fori_loop(0, 4, mrow, 0)
                sidx[pl.ds(fc, 16)] = lv
                return fc + 16

            return lax.cond(lv[0] == lv[15], uniform, mixed, fc)

        return lax.fori_loop(0, GROUPS, group, fc)

    issue(0, x0, idx0, sem_x0, sem_i0)
    issue(1, x1, idx1, sem_x1, sem_i1)

    def pair(p, fc):
        j0 = 2 * p
        wait(j0, x0, idx0, sem_x0, sem_i0)
        fc = process(j0, x0, idx0, fc)

        @pl.when(j0 + 2 < NCHUNK)
        def _():
            issue(j0 + 2, x0, idx0, sem_x0, sem_i0)

        j1 = j0 + 1
        wait(j1, x1, idx1, sem_x1, sem_i1)
        fc = process(j1, x1, idx1, fc)

        @pl.when(j1 + 2 < NCHUNK)
        def _():
            issue(j1 + 2, x1, idx1, sem_x1, sem_i1)

        return fc

    fc = lax.fori_loop(0, NCHUNK // 2, pair, jnp.int32(0))
    if NCHUNK % 2:
        j = NCHUNK - 1
        wait(j, x0, idx0, sem_x0, sem_i0)
        fc = process(j, x0, idx0, fc)
    flush_scatter()
    plsc.subcore_barrier()

    pltpu.sync_copy(acc.at[pl.ds(base, CROWS), pl.ds(0, HALF)],
                    total_hbm.at[pl.ds(base, CROWS), pl.ds(ci * HALF, HALF)])
    pltpu.sync_copy(acc.at[pl.ds(base, CROWS), pl.ds(HALF, HALF)],
                    total2_hbm.at[pl.ds(base, CROWS), pl.ds(ci * HALF, HALF)])

    @pl.when(ci == 0)
    def _():
        pltpu.sync_copy(acc.at[pl.ds(base, CROWS), pl.ds(2 * HALF, 16)],
                        counts_hbm.at[pl.ds(base, CROWS)])


@jax.jit
def _sc_segment_stats(inputs, labels3):
    mesh = plsc.VectorSubcoreMesh(core_axis_name="c", subcore_axis_name="s")
    f = pl.kernel(
        _sc_body,
        out_type=(
            jax.ShapeDtypeStruct((C, D), jnp.float32),
            jax.ShapeDtypeStruct((C, D), jnp.float32),
            jax.ShapeDtypeStruct((C, 16), jnp.float32),
        ),
        mesh=mesh,
        compiler_params=pltpu.CompilerParams(use_tc_tiling_on_sc=False),
        scratch_types=[
            pltpu.VMEM((CHUNK,), jnp.int32),
            pltpu.VMEM((CHUNK,), jnp.int32),
            pltpu.VMEM((CHUNK, HALF), jnp.float32),
            pltpu.VMEM((CHUNK, HALF), jnp.float32),
            pltpu.VMEM((STAGE, W), jnp.float32),
            pltpu.VMEM((STAGE,), jnp.int32),
            pltpu.SemaphoreType.DMA,
            pltpu.SemaphoreType.DMA,
            pltpu.SemaphoreType.DMA,
            pltpu.SemaphoreType.DMA,
            pltpu.VMEM_SHARED((C + TRASH, W), jnp.float32),
        ],
    )
    return f(inputs, labels3)


def _update_body(total_ref, total2_ref, counts_ref, mean_ref, var_ref, cc_ref,
                 new_mean_ref, new_var_ref, new_cc_ref):
    cnt = counts_ref[:, 0:1]
    cc_f = cc_ref[...].astype(jnp.float32)
    inv = 1.0 / (cc_f + cnt)
    keep = cc_f * inv
    new_mean = mean_ref[...] * keep + total_ref[...] * inv
    new_mean_ref[...] = new_mean
    new_var_ref[...] = var_ref[...] * keep + (
        total2_ref[...] - cnt * new_mean * new_mean) * inv
    new_cc_ref[...] = cc_ref[...] + cnt.astype(jnp.int32)


@jax.jit
def _tc_update(total, total2, counts, running_mean, running_var, class_count):
    BC = 1000
    grid = C // BC
    return pl.pallas_call(
        _update_body,
        grid=(grid,),
        in_specs=[
            pl.BlockSpec((BC, D), lambda i: (i, 0)),
            pl.BlockSpec((BC, D), lambda i: (i, 0)),
            pl.BlockSpec((BC, 16), lambda i: (i, 0)),
            pl.BlockSpec((BC, D), lambda i: (i, 0)),
            pl.BlockSpec((BC, D), lambda i: (i, 0)),
            pl.BlockSpec((BC, 1), lambda i: (i, 0)),
        ],
        out_specs=[
            pl.BlockSpec((BC, D), lambda i: (i, 0)),
            pl.BlockSpec((BC, D), lambda i: (i, 0)),
            pl.BlockSpec((BC, 1), lambda i: (i, 0)),
        ],
        out_shape=[
            jax.ShapeDtypeStruct((C, D), jnp.float32),
            jax.ShapeDtypeStruct((C, D), jnp.float32),
            jax.ShapeDtypeStruct((C, 1), jnp.int32),
        ],
    )(total, total2, counts, running_mean, running_var, class_count)


def kernel(inputs, labels, running_mean, running_var, class_count):
    labels3 = labels.reshape(NS, NCHUNK, CHUNK)
    total, total2, counts = _sc_segment_stats(inputs, labels3)
    new_mean, new_var, new_cc = _tc_update(
        total, total2, counts, running_mean, running_var, class_count)
    return new_mean, new_var, new_cc


# speedup vs baseline: 1.2563x; 1.2563x over previous
import jax
import jax.numpy as jnp
from jax import lax
from jax.experimental import pallas as pl
from jax.experimental.pallas import tpu as pltpu
from jax.experimental.pallas import tpu_sc as plsc

N = 320000
D = 128
C = 10000

NC = 2
NS = 16
HALF = D // NC
ROWS_PER_SUB = N // NS
CHUNK = 80
NCHUNK = ROWS_PER_SUB // CHUNK
GROUPS = CHUNK // 16
STAGE = CHUNK
W = 2 * HALF + 16
CROWS = C // NS


def _ds16(c):
    return pl.ds(16 * c, 16)


def _tree(vals):
    while len(vals) > 1:
        nxt = [a + b for a, b in zip(vals[::2], vals[1::2])]
        if len(vals) % 2:
            nxt[-1] = nxt[-1] + vals[-1]
        vals = nxt
    return vals[0]


def _sc_body(inputs_hbm, labels_hbm, total_hbm, total2_hbm, counts_hbm,
             idx0, idx1, x0, x1, stage0, stage1, sidx0, sidx1,
             sem_x0, sem_x1, sem_i0, sem_i1, sem_s0, sem_s1, acc):
    ci = lax.axis_index("c")
    si = lax.axis_index("s")

    zeros16 = jnp.zeros((16,), jnp.float32)
    lane0 = lax.iota(jnp.int32, 16) == 0
    one_pat = jnp.where(lane0, 1.0, 0.0)
    sixteen_pat = jnp.where(lane0, 16.0, 0.0)
    trash_base = jnp.full((16,), C, jnp.int32) + lax.iota(jnp.int32, 16)

    def sidx_trash(sidx_v):
        for k in range(STAGE // 16):
            sidx_v[_ds16(k)] = trash_base + 16 * k

    @plsc.parallel_loop(0, STAGE, unroll=8)
    def _zs(r):
        for c in range(W // 16):
            stage0[r, _ds16(c)] = zeros16

    sidx_trash(sidx0)
    sidx_trash(sidx1)

    base = si * CROWS
    for z in range(CROWS // STAGE):
        pltpu.sync_copy(stage0, acc.at[pl.ds(base + z * STAGE, STAGE)])
    rem = CROWS - (CROWS // STAGE) * STAGE
    zb = base + (CROWS // STAGE) * STAGE
    pltpu.sync_copy(stage0.at[pl.ds(0, rem)], acc.at[pl.ds(zb, rem)])

    @pl.when(si == NS - 1)
    def _():
        pltpu.sync_copy(stage0, acc.at[pl.ds(C, STAGE)])

    plsc.subcore_barrier()

    pltpu.async_copy(stage0, acc.at[sidx0], sem_s0, add=True)
    pltpu.async_copy(stage1, acc.at[sidx1], sem_s1, add=True)

    def x_src(j):
        row0 = si * ROWS_PER_SUB + j * CHUNK
        return inputs_hbm.at[pl.ds(row0, CHUNK), pl.ds(ci * HALF, HALF)]

    def issue(j, x_v, idx_v, sem_x, sem_i):
        pltpu.async_copy(x_src(j), x_v, sem_x)
        pltpu.async_copy(labels_hbm.at[si, j], idx_v, sem_i)

    def wait(j, x_v, idx_v, sem_x, sem_i):
        pltpu.make_async_copy(x_src(j), x_v, sem_x).wait()
        pltpu.make_async_copy(labels_hbm.at[si, j], idx_v, sem_i).wait()

    def process(x_v, idx_v, stage, sidx):
        def group(g, fc):
            r0 = 16 * g
            lv = idx_v[pl.ds(r0, 16)]

            def uniform(fc):
                for c in range(HALF // 16):
                    vs = [x_v[r0 + r, _ds16(c)] for r in range(16)]
                    sq = [v * v for v in vs]
                    stage[fc, _ds16(c)] = _tree(vs)
                    stage[fc, _ds16(4 + c)] = _tree(sq)
                stage[fc, _ds16(8)] = sixteen_pat
                old = sidx[pl.ds(fc, 16)]
                sidx[pl.ds(fc, 16)] = jnp.where(lane0, lv, old)
                return fc + 1

            def mixed(fc):
                for r in range(16):
                    k = fc + r
                    for c in range(HALF // 16):
                        v = x_v[r0 + r, _ds16(c)]
                        stage[k, _ds16(c)] = v
                        stage[k, _ds16(4 + c)] = v * v
                    stage[k, _ds16(8)] = one_pat
                sidx[pl.ds(fc, 16)] = lv
                return fc + 16

            return lax.cond(lv[0] == lv[15], uniform, mixed, fc)

        lax.fori_loop(0, GROUPS, group, jnp.int32(0))

    issue(0, x0, idx0, sem_x0, sem_i0)
    issue(1, x1, idx1, sem_x1, sem_i1)

    def step(j, x_v, idx_v, stage, sidx, sem_x, sem_i, sem_s):
        wait(j, x_v, idx_v, sem_x, sem_i)
        pltpu.make_async_copy(stage, acc.at[sidx], sem_s).wait()
        sidx_trash(sidx)
        process(x_v, idx_v, stage, sidx)
        pltpu.async_copy(stage, acc.at[sidx], sem_s, add=True)

        @pl.when(j + 2 < NCHUNK)
        def _():
            issue(j + 2, x_v, idx_v, sem_x, sem_i)

    def pair(p, carry):
        j0 = 2 * p
        step(j0, x0, idx0, stage0, sidx0, sem_x0, sem_i0, sem_s0)
        step(j0 + 1, x1, idx1, stage1, sidx1, sem_x1, sem_i1, sem_s1)
        return carry

    lax.fori_loop(0, NCHUNK // 2, pair, jnp.int32(0))
    pltpu.make_async_copy(stage0, acc.at[sidx0], sem_s0).wait()
    pltpu.make_async_copy(stage1, acc.at[sidx1], sem_s1).wait()
    plsc.subcore_barrier()

    pltpu.sync_copy(acc.at[pl.ds(base, CROWS), pl.ds(0, HALF)],
                    total_hbm.at[pl.ds(base, CROWS), pl.ds(ci * HALF, HALF)])
    pltpu.sync_copy(acc.at[pl.ds(base, CROWS), pl.ds(HALF, HALF)],
                    total2_hbm.at[pl.ds(base, CROWS), pl.ds(ci * HALF, HALF)])

    @pl.when(ci == 0)
    def _():
        pltpu.sync_copy(acc.at[pl.ds(base, CROWS), pl.ds(2 * HALF, 16)],
                        counts_hbm.at[pl.ds(base, CROWS)])


@jax.jit
def _sc_segment_stats(inputs, labels3):
    mesh = plsc.VectorSubcoreMesh(core_axis_name="c", subcore_axis_name="s")
    f = pl.kernel(
        _sc_body,
        out_type=(
            jax.ShapeDtypeStruct((C, D), jnp.float32),
            jax.ShapeDtypeStruct((C, D), jnp.float32),
            jax.ShapeDtypeStruct((C, 16), jnp.float32),
        ),
        mesh=mesh,
        compiler_params=pltpu.CompilerParams(use_tc_tiling_on_sc=False),
        scratch_types=[
            pltpu.VMEM((CHUNK,), jnp.int32),
            pltpu.VMEM((CHUNK,), jnp.int32),
            pltpu.VMEM((CHUNK, HALF), jnp.float32),
            pltpu.VMEM((CHUNK, HALF), jnp.float32),
            pltpu.VMEM((STAGE, W), jnp.float32),
            pltpu.VMEM((STAGE, W), jnp.float32),
            pltpu.VMEM((STAGE,), jnp.int32),
            pltpu.VMEM((STAGE,), jnp.int32),
            pltpu.SemaphoreType.DMA,
            pltpu.SemaphoreType.DMA,
            pltpu.SemaphoreType.DMA,
            pltpu.SemaphoreType.DMA,
            pltpu.SemaphoreType.DMA,
            pltpu.SemaphoreType.DMA,
            pltpu.VMEM_SHARED((C + STAGE, W), jnp.float32),
        ],
    )
    return f(inputs, labels3)


def _update_body(total_ref, total2_ref, counts_ref, mean_ref, var_ref, cc_ref,
                 new_mean_ref, new_var_ref, new_cc_ref):
    cnt = counts_ref[:, 0:1]
    cc_f = cc_ref[...].astype(jnp.float32)
    inv = 1.0 / (cc_f + cnt)
    keep = cc_f * inv
    new_mean = mean_ref[...] * keep + total_ref[...] * inv
    new_mean_ref[...] = new_mean
    new_var_ref[...] = var_ref[...] * keep + (
        total2_ref[...] - cnt * new_mean * new_mean) * inv
    new_cc_ref[...] = cc_ref[...] + cnt.astype(jnp.int32)


@jax.jit
def _tc_update(total, total2, counts, running_mean, running_var, class_count):
    BC = 1000
    grid = C // BC
    return pl.pallas_call(
        _update_body,
        grid=(grid,),
        in_specs=[
            pl.BlockSpec((BC, D), lambda i: (i, 0)),
            pl.BlockSpec((BC, D), lambda i: (i, 0)),
            pl.BlockSpec((BC, 16), lambda i: (i, 0)),
            pl.BlockSpec((BC, D), lambda i: (i, 0)),
            pl.BlockSpec((BC, D), lambda i: (i, 0)),
            pl.BlockSpec((BC, 1), lambda i: (i, 0)),
        ],
        out_specs=[
            pl.BlockSpec((BC, D), lambda i: (i, 0)),
            pl.BlockSpec((BC, D), lambda i: (i, 0)),
            pl.BlockSpec((BC, 1), lambda i: (i, 0)),
        ],
        out_shape=[
            jax.ShapeDtypeStruct((C, D), jnp.float32),
            jax.ShapeDtypeStruct((C, D), jnp.float32),
            jax.ShapeDtypeStruct((C, 1), jnp.int32),
        ],
    )(total, total2, counts, running_mean, running_var, class_count)


def kernel(inputs, labels, running_mean, running_var, class_count):
    labels3 = labels.reshape(NS, NCHUNK, CHUNK)
    total, total2, counts = _sc_segment_stats(inputs, labels3)
    new_mean, new_var, new_cc = _tc_update(
        total, total2, counts, running_mean, running_var, class_count)
    return new_mean, new_var, new_cc
